# Initial kernel scaffold; baseline (speedup 1.0000x reference)
import jax
import jax.numpy as jnp
from jax.experimental import pallas as pl


def kernel(features, W0, W1, Wb, bb, Hparam, edge_index_0, edge_index_1, perm):
    raise NotImplementedError("write your pallas kernel here")


# speedup vs baseline: 3.3092x; 3.3092x over previous
import functools

import jax
import jax.numpy as jnp
from jax import lax
from jax.experimental import pallas as pl
from jax.experimental.pallas import tpu as pltpu
from jax.experimental.pallas import tpu_sc as plsc

N = 10000
F = 128
H = 128
V = 2
E = 320000
REG_COEF = 0.001

NC = 2
NS = 16
L = 16
TILE_EDGES = E // NS
CHUNK = 80
NCHUNK = TILE_EDGES // CHUNK
ROWS_PER_TILE = 632


def _mm_body(x_ref, w_ref, out_ref):
    out_ref[0] = jnp.dot(x_ref[...], w_ref[0], preferred_element_type=jnp.float32)


def _xw(features, Wstack):
    return pl.pallas_call(
        _mm_body,
        grid=(V,),
        in_specs=[
            pl.BlockSpec((N, F), lambda i: (0, 0)),
            pl.BlockSpec((1, F, H), lambda i: (i, 0, 0)),
        ],
        out_specs=pl.BlockSpec((1, N, H), lambda i: (i, 0, 0)),
        out_shape=jax.ShapeDtypeStruct((V, N, H), jnp.float32),
    )(features, Wstack)


PROWS_PER_TILE = 640
PCHUNK = 128
NPCHUNK = PROWS_PER_TILE // PCHUNK


def _sc_body(xw, src, dst, perm_hbm, zeros,
             out, xwp,
             pidx_v, gidx_v, didx_v, rows_v, prow_v, acc, sem):
    c = lax.axis_index("c")
    s = lax.axis_index("s")
    coff = c * N

    prow0 = jnp.minimum(s * PROWS_PER_TILE, N - PROWS_PER_TILE)
    for k in range(NPCHUNK):
        rbase = prow0 + k * PCHUNK
        pltpu.sync_copy(perm_hbm.at[pl.ds(rbase, PCHUNK)], pidx_v)
        for i in range(PCHUNK // L):
            sl = pl.ds(i * L, L)
            pidx_v[sl] = pidx_v[sl] + coff
        pltpu.async_copy(xw.at[pidx_v], prow_v, sem).wait()
        pltpu.sync_copy(prow_v, xwp.at[pl.ds(coff + rbase, PCHUNK)])
    plsc.subcore_barrier()

    row0 = jnp.minimum(s * ROWS_PER_TILE, N - ROWS_PER_TILE)
    ebase = c * E + s * TILE_EDGES

    for phase, table in ((0, xw), (1, xwp)):
        pltpu.sync_copy(zeros.at[pl.ds(row0, ROWS_PER_TILE)],
                        acc.at[pl.ds(row0, ROWS_PER_TILE)])
        plsc.subcore_barrier()

        def chunk(j, carry):
            base = ebase + j * CHUNK
            pltpu.sync_copy(src.at[pl.ds(base, CHUNK)], gidx_v)
            pltpu.sync_copy(dst.at[pl.ds(base, CHUNK)], didx_v)
            for i in range(CHUNK // L):
                sl = pl.ds(i * L, L)
                gidx_v[sl] = gidx_v[sl] + coff
            pltpu.async_copy(table.at[gidx_v], rows_v, sem).wait()
            pltpu.sync_copy(rows_v, acc.at[didx_v], add=True)
            return carry

        lax.fori_loop(0, NCHUNK, chunk, 0)
        plsc.subcore_barrier()
        outbase = (phase * V + c) * N + row0
        pltpu.sync_copy(acc.at[pl.ds(row0, ROWS_PER_TILE)],
                        out.at[pl.ds(outbase, ROWS_PER_TILE)])
        plsc.subcore_barrier()


def _segment_sums(xw_flat, src_flat, dst_flat, perm, zeros):
    mesh = plsc.VectorSubcoreMesh(core_axis_name="c", subcore_axis_name="s")
    f = functools.partial(
        pl.kernel,
        mesh=mesh,
        out_type=(jax.ShapeDtypeStruct((2 * V * N, H), jnp.float32),
                  jax.ShapeDtypeStruct((V * N, H), jnp.float32)),
        scratch_types=[
            pltpu.VMEM((PCHUNK,), jnp.int32),
            pltpu.VMEM((CHUNK,), jnp.int32),
            pltpu.VMEM((CHUNK,), jnp.int32),
            pltpu.VMEM((CHUNK, H), jnp.float32),
            pltpu.VMEM((PCHUNK, H), jnp.float32),
            pltpu.VMEM_SHARED((N, H), jnp.float32),
            pltpu.SemaphoreType.DMA,
        ],
    )(_sc_body)
    sums, _ = f(xw_flat, src_flat, dst_flat, perm, zeros)
    return sums


def _loss_body(s_ref, wb_ref, bb_ref, hp_ref, out_ref):
    bb0 = bb_ref[0]
    xent = jnp.float32(0.0)
    hs = []
    for k in range(2 * V):
        hs.append(jnp.maximum(s_ref[k], 0.0))
    for v in range(V):
        h1, h2 = hs[v], hs[V + v]
        cvec = 1.0 / (1.0 + jnp.exp(-jnp.mean(h1, axis=0, keepdims=True)))
        w = jnp.sum(wb_ref[...] * cvec, axis=1, keepdims=True)
        s1 = jnp.dot(h1, w, preferred_element_type=jnp.float32) + bb0
        s2 = jnp.dot(h2, w, preferred_element_type=jnp.float32) + bb0
        t1 = jnp.maximum(s1, 0.0) - s1 + jnp.log1p(jnp.exp(-jnp.abs(s1)))
        t2 = jnp.maximum(s2, 0.0) + jnp.log1p(jnp.exp(-jnp.abs(s2)))
        xent = xent + (jnp.sum(t1) + jnp.sum(t2)) / jnp.float32(2 * N)
    h1a = 0.5 * (hs[0] + hs[1])
    h2a = 0.5 * (hs[2] + hs[3])
    hp = hp_ref[...]
    pos = jnp.sum((hp - h1a) ** 2)
    neg = jnp.sum((hp - h2a) ** 2)
    total = xent + jnp.float32(REG_COEF) * (pos - neg)
    out_ref[...] = jnp.reshape(total, (1, 1))


def _loss(sums, Wb, bb, Hparam):
    return pl.pallas_call(
        _loss_body,
        out_shape=jax.ShapeDtypeStruct((1, 1), jnp.float32),
    )(sums, Wb, bb, Hparam)


def kernel(features, W0, W1, Wb, bb, Hparam, edge_index_0, edge_index_1, perm):
    xw = _xw(features, jnp.stack([W0, W1]))
    xw_flat = xw.reshape(V * N, H)
    src_flat = jnp.concatenate([edge_index_0[0], edge_index_1[0]])
    dst_flat = jnp.concatenate([edge_index_0[1], edge_index_1[1]])
    zeros = jnp.zeros((N, H), jnp.float32)
    sums = _segment_sums(xw_flat, src_flat, dst_flat, perm, zeros)
    loss = _loss(sums.reshape(2 * V, N, H), Wb, bb, Hparam)
    return loss.reshape(())


# speedup vs baseline: 6.2009x; 1.8738x over previous
import functools

import jax
import jax.numpy as jnp
from jax import lax
from jax.experimental import pallas as pl
from jax.experimental.pallas import tpu as pltpu
from jax.experimental.pallas import tpu_sc as plsc

N = 10000
F = 128
H = 128
V = 2
E = 320000
REG_COEF = 0.001

NC = 2
NS = 16
L = 16
TILE_EDGES = E // NS
CHUNK = 80
NCHUNK = TILE_EDGES // CHUNK
ROWS_PER_TILE = 632


def _mm_body(x_ref, w_ref, out_ref):
    out_ref[0] = jnp.dot(x_ref[...], w_ref[0], preferred_element_type=jnp.float32)


def _xw(features, Wstack):
    return pl.pallas_call(
        _mm_body,
        grid=(V,),
        in_specs=[
            pl.BlockSpec((N, F), lambda i: (0, 0)),
            pl.BlockSpec((1, F, H), lambda i: (i, 0, 0)),
        ],
        out_specs=pl.BlockSpec((1, N, H), lambda i: (i, 0, 0)),
        out_shape=jax.ShapeDtypeStruct((V, N, H), jnp.float32),
    )(features, Wstack)


PROWS_PER_TILE = 640
NPB = PROWS_PER_TILE // CHUNK


def _sc_body(xw, src, dst, perm_hbm, zeros,
             out, xwp,
             sidx, didx1d, didx_s, rows, acc, gsem, ssem, isem):
    c = lax.axis_index("c")
    s = lax.axis_index("s")

    prow0 = jnp.minimum(s * PROWS_PER_TILE, N - PROWS_PER_TILE)
    coff = c * N
    for b in range(NPB):
        p = b & 1
        pltpu.sync_copy(perm_hbm.at[pl.ds(coff + prow0 + b * CHUNK, CHUNK)],
                        sidx[p])
        pltpu.async_copy(xw.at[sidx[p]], rows[p], gsem[p]).wait()
        pltpu.async_copy(rows[p],
                         xwp.at[pl.ds(coff + prow0 + b * CHUNK, CHUNK)],
                         ssem[p]).wait()
    plsc.subcore_barrier()

    row0 = jnp.minimum(s * ROWS_PER_TILE, N - ROWS_PER_TILE)
    ebase = c * E + s * TILE_EDGES
    pltpu.sync_copy(dst.at[pl.ds(ebase, TILE_EDGES)], didx1d)

    for phase, table in ((0, xw), (1, xwp)):
        pltpu.sync_copy(zeros.at[pl.ds(row0, ROWS_PER_TILE)],
                        acc.at[pl.ds(row0, ROWS_PER_TILE)])
        plsc.subcore_barrier()

        pltpu.sync_copy(src.at[pl.ds(ebase, CHUNK)], sidx[0])
        pltpu.async_copy(table.at[sidx[0]], rows[0], gsem[0])
        pltpu.async_copy(src.at[pl.ds(ebase + CHUNK, CHUNK)], sidx[1], isem[1])

        def pair(t, carry):
            for p in (0, 1):
                m = 2 * t + p
                base = ebase + m * CHUNK
                pltpu.make_async_copy(table.at[sidx[p]], rows[p],
                                      gsem[p]).wait()
                for i in range(CHUNK // L):
                    sl = pl.ds(i * L, L)
                    didx_s[p][sl] = didx1d[pl.ds(m * CHUNK + i * L, L)]
                pltpu.async_copy(rows[p], acc.at[didx_s[p]], ssem[p],
                                 add=True)

                @pl.when(m < NCHUNK - 1)
                def _():
                    pltpu.make_async_copy(
                        src.at[pl.ds(base + CHUNK, CHUNK)], sidx[1 - p],
                        isem[1 - p]).wait()

                @pl.when(m > 0)
                def _():
                    pltpu.make_async_copy(rows[1 - p], acc.at[didx_s[1 - p]],
                                          ssem[1 - p]).wait()

                @pl.when(m < NCHUNK - 1)
                def _():
                    pltpu.async_copy(table.at[sidx[1 - p]], rows[1 - p],
                                     gsem[1 - p])

                @pl.when(m < NCHUNK - 2)
                def _():
                    pltpu.async_copy(
                        src.at[pl.ds(base + 2 * CHUNK, CHUNK)], sidx[p],
                        isem[p])
            return carry

        lax.fori_loop(0, NCHUNK // 2, pair, 0)
        pltpu.make_async_copy(rows[1], acc.at[didx_s[1]], ssem[1]).wait()
        plsc.subcore_barrier()
        outbase = (phase * V + c) * N + row0
        pltpu.sync_copy(acc.at[pl.ds(row0, ROWS_PER_TILE)],
                        out.at[pl.ds(outbase, ROWS_PER_TILE)])
        plsc.subcore_barrier()


def _segment_sums(xw_flat, srcv, dst_flat, permv, zeros):
    mesh = plsc.VectorSubcoreMesh(core_axis_name="c", subcore_axis_name="s")
    f = functools.partial(
        pl.kernel,
        mesh=mesh,
        out_type=(jax.ShapeDtypeStruct((2 * V * N, H), jnp.float32),
                  jax.ShapeDtypeStruct((V * N, H), jnp.float32)),
        scratch_types=[
            [pltpu.VMEM((CHUNK,), jnp.int32) for _ in range(2)],
            pltpu.VMEM((TILE_EDGES,), jnp.int32),
            [pltpu.VMEM((CHUNK,), jnp.int32) for _ in range(2)],
            [pltpu.VMEM((CHUNK, H), jnp.float32) for _ in range(2)],
            pltpu.VMEM_SHARED((N, H), jnp.float32),
            [pltpu.SemaphoreType.DMA for _ in range(2)],
            [pltpu.SemaphoreType.DMA for _ in range(2)],
            [pltpu.SemaphoreType.DMA for _ in range(2)],
        ],
    )(_sc_body)
    sums, _ = f(xw_flat, srcv, dst_flat, permv, zeros)
    return sums


def _loss_body(s_ref, wb_ref, bb_ref, hp_ref, out_ref):
    bb0 = bb_ref[0]
    xent = jnp.float32(0.0)
    hs = []
    for k in range(2 * V):
        hs.append(jnp.maximum(s_ref[k], 0.0))
    for v in range(V):
        h1, h2 = hs[v], hs[V + v]
        cvec = 1.0 / (1.0 + jnp.exp(-jnp.mean(h1, axis=0, keepdims=True)))
        w = jnp.sum(wb_ref[...] * cvec, axis=1, keepdims=True)
        s1 = jnp.dot(h1, w, preferred_element_type=jnp.float32) + bb0
        s2 = jnp.dot(h2, w, preferred_element_type=jnp.float32) + bb0
        t1 = jnp.maximum(s1, 0.0) - s1 + jnp.log1p(jnp.exp(-jnp.abs(s1)))
        t2 = jnp.maximum(s2, 0.0) + jnp.log1p(jnp.exp(-jnp.abs(s2)))
        xent = xent + (jnp.sum(t1) + jnp.sum(t2)) / jnp.float32(2 * N)
    h1a = 0.5 * (hs[0] + hs[1])
    h2a = 0.5 * (hs[2] + hs[3])
    hp = hp_ref[...]
    pos = jnp.sum((hp - h1a) ** 2)
    neg = jnp.sum((hp - h2a) ** 2)
    total = xent + jnp.float32(REG_COEF) * (pos - neg)
    out_ref[...] = jnp.reshape(total, (1, 1))


def _loss(sums, Wb, bb, Hparam):
    return pl.pallas_call(
        _loss_body,
        out_shape=jax.ShapeDtypeStruct((1, 1), jnp.float32),
    )(sums, Wb, bb, Hparam)


def kernel(features, W0, W1, Wb, bb, Hparam, edge_index_0, edge_index_1, perm):
    xw = _xw(features, jnp.stack([W0, W1]))
    xw_flat = xw.reshape(V * N, H)
    src_v = jnp.concatenate([edge_index_0[0], edge_index_1[0] + N])
    dst_flat = jnp.concatenate([edge_index_0[1], edge_index_1[1]])
    perm_v = jnp.concatenate([perm, perm + N])
    zeros = jnp.zeros((N, H), jnp.float32)
    sums = _segment_sums(xw_flat, src_v, dst_flat, perm_v, zeros)
    loss = _loss(sums.reshape(2 * V, N, H), Wb, bb, Hparam)
    return loss.reshape(())
